# Initial kernel scaffold; baseline (speedup 1.0000x reference)
import jax
import jax.numpy as jnp
from jax.experimental import pallas as pl


def kernel(embed_ids, sentence_len, parents, emb_matrix, W_ih_f, W_hh_f, b_ih_f, b_hh_f, W_ih_b, W_hh_b, b_ih_b, b_hh_b, W_iou, b_iou, U_iou, U_f, b_Uf, W_hid, b_hid):
    raise NotImplementedError("write your pallas kernel here")


# speedup vs baseline: 120.0072x; 120.0072x over previous
import functools

import jax
import jax.numpy as jnp
from jax import lax
from jax.experimental import pallas as pl
from jax.experimental.pallas import tpu as pltpu
from jax.experimental.pallas import tpu_sc as plsc

B, L, EMB, H, D = 16, 512, 300, 256, 256
EMB_P = 384
N = B * L
G4 = 4 * H
IOU = 3 * D
MSG = 4 * D
HIGHEST = lax.Precision.HIGHEST


def _gather_embeddings(emb_matrix, ids_tm):
    info = plsc.get_sparse_core_info()
    nw = info.num_cores * info.num_subcores
    per_w = N // nw
    K = 128
    nk = per_w // K
    idx3 = ids_tm.reshape(nw, nk, K)
    mesh = plsc.VectorSubcoreMesh(core_axis_name="c", subcore_axis_name="s")

    @functools.partial(
        pl.kernel, mesh=mesh,
        out_type=jax.ShapeDtypeStruct((N, EMB_P), jnp.float32),
        scratch_types=[
            pltpu.VMEM((nk, K), jnp.int32),
            pltpu.VMEM((per_w, EMB_P), jnp.float32),
            pltpu.SemaphoreType.DMA,
        ],
    )
    def gather_k(table_hbm, idx_hbm, out_hbm, idx_v, rows_v, sem):
        wid = lax.axis_index("s") * info.num_cores + lax.axis_index("c")
        pltpu.sync_copy(idx_hbm.at[wid], idx_v)
        for j in range(nk):
            pltpu.async_copy(
                table_hbm.at[idx_v.at[j]], rows_v.at[pl.ds(j * K, K)], sem
            ).wait()
        pltpu.sync_copy(rows_v, out_hbm.at[pl.ds(wid * per_w, per_w)])

    return gather_k(emb_matrix, idx3)


_C = 64
_NT = L // _C


def _lstm_body(emb_f_ref, emb_b_ref, wif_ref, whf_ref, bf_ref,
               wib_ref, whb_ref, bb_ref, hf_out, hb_out,
               xf_ref, xb_ref, hf_c, cf_c, hb_c, cb_c):
    i = pl.program_id(0)

    @pl.when(i == 0)
    def _():
        hf_c[...] = jnp.zeros_like(hf_c)
        cf_c[...] = jnp.zeros_like(cf_c)
        hb_c[...] = jnp.zeros_like(hb_c)
        cb_c[...] = jnp.zeros_like(cb_c)

    ef = emb_f_ref[...].reshape(_C * B, EMB_P)
    xf_ref[...] = jnp.dot(ef, wif_ref[...], preferred_element_type=jnp.float32).reshape(_C, B, G4)
    eb = emb_b_ref[...].reshape(_C * B, EMB_P)
    xb_ref[...] = jnp.dot(eb, wib_ref[...], preferred_element_type=jnp.float32).reshape(_C, B, G4)

    def cell(g, c):
        ig = g[:, 0:H]
        fg = g[:, H:2 * H]
        gg = g[:, 2 * H:3 * H]
        og = g[:, 3 * H:4 * H]
        c_new = jax.nn.sigmoid(fg) * c + jax.nn.sigmoid(ig) * jnp.tanh(gg)
        h_new = jax.nn.sigmoid(og) * jnp.tanh(c_new)
        return h_new, c_new

    def step(j, _):
        gf = xf_ref[j] + jnp.dot(hf_c[...], whf_ref[...], preferred_element_type=jnp.float32) + bf_ref[...]
        hf, cf = cell(gf, cf_c[...])
        hf_c[...] = hf
        cf_c[...] = cf
        hf_out[j] = hf
        r = _C - 1 - j
        gb = xb_ref[r] + jnp.dot(hb_c[...], whb_ref[...], preferred_element_type=jnp.float32) + bb_ref[...]
        hb, cb = cell(gb, cb_c[...])
        hb_c[...] = hb
        cb_c[...] = cb
        hb_out[r] = hb
        return 0

    lax.fori_loop(0, _C, step, 0)


def _bilstm(embeds, wif, whf, bf, wib, whb, bb):
    return pl.pallas_call(
        _lstm_body,
        grid=(_NT,),
        in_specs=[
            pl.BlockSpec((_C, B, EMB_P), lambda i: (i, 0, 0)),
            pl.BlockSpec((_C, B, EMB_P), lambda i: (_NT - 1 - i, 0, 0)),
            pl.BlockSpec((EMB_P, G4), lambda i: (0, 0)),
            pl.BlockSpec((H, G4), lambda i: (0, 0)),
            pl.BlockSpec((1, G4), lambda i: (0, 0)),
            pl.BlockSpec((EMB_P, G4), lambda i: (0, 0)),
            pl.BlockSpec((H, G4), lambda i: (0, 0)),
            pl.BlockSpec((1, G4), lambda i: (0, 0)),
        ],
        out_specs=[
            pl.BlockSpec((_C, B, H), lambda i: (i, 0, 0)),
            pl.BlockSpec((_C, B, H), lambda i: (_NT - 1 - i, 0, 0)),
        ],
        out_shape=[
            jax.ShapeDtypeStruct((L, B, H), jnp.float32),
            jax.ShapeDtypeStruct((L, B, H), jnp.float32),
        ],
        scratch_shapes=[
            pltpu.VMEM((_C, B, G4), jnp.float32),
            pltpu.VMEM((_C, B, G4), jnp.float32),
            pltpu.VMEM((B, H), jnp.float32),
            pltpu.VMEM((B, H), jnp.float32),
            pltpu.VMEM((B, H), jnp.float32),
            pltpu.VMEM((B, H), jnp.float32),
        ],
    )(embeds, embeds, wif, whf, bf, wib, whb, bb)


def _tree_body(plocal_ref, hf_ref, hb_ref, wiou_ref, biou_ref, ucat_ref,
               buf_ref, hout_ref, acc_ref):
    i = pl.program_id(0)

    @pl.when(i == 0)
    def _():
        acc_ref[...] = jnp.zeros_like(acc_ref)

    x = jnp.concatenate([hf_ref[...], hb_ref[...]], axis=-1).reshape(_C * B, 2 * H)
    iou0 = (jnp.dot(x, wiou_ref[...], preferred_element_type=jnp.float32) + biou_ref[...]).reshape(_C, B, IOU)
    acc_ref[pl.ds(i * _C, _C), :, 0:IOU] += iou0

    def step(j, _):
        v = i * _C + j
        row = acc_ref[v]
        iou = row[:, 0:IOU]
        c_prev = row[:, IOU:MSG]
        ig = iou[:, 0:D]
        og = iou[:, D:2 * D]
        ug = iou[:, 2 * D:3 * D]
        c_v = jax.nn.sigmoid(ig) * jnp.tanh(ug) + c_prev
        h_v = jax.nn.sigmoid(og) * jnp.tanh(c_v)
        hout_ref[j] = h_v
        m = jnp.dot(h_v, ucat_ref[...], preferred_element_type=jnp.float32)
        f = jax.nn.sigmoid(m[:, IOU:MSG] + buf_ref[...])
        msg = jnp.concatenate([m[:, 0:IOU], f * c_v], axis=-1)
        for b in range(B):
            p = plocal_ref[v, b]
            acc_ref[p, b:b + 1, :] += msg[b:b + 1, :]
        return 0

    lax.fori_loop(0, _C, step, 0)


def _tree_lstm(plocal, hf, hb, wiou, biou, ucat, buf):
    return pl.pallas_call(
        _tree_body,
        grid=(_NT,),
        in_specs=[
            pl.BlockSpec(memory_space=pltpu.SMEM),
            pl.BlockSpec((_C, B, H), lambda i: (i, 0, 0)),
            pl.BlockSpec((_C, B, H), lambda i: (i, 0, 0)),
            pl.BlockSpec((2 * H, IOU), lambda i: (0, 0)),
            pl.BlockSpec((1, IOU), lambda i: (0, 0)),
            pl.BlockSpec((D, MSG), lambda i: (0, 0)),
            pl.BlockSpec((1, D), lambda i: (0, 0)),
        ],
        out_specs=pl.BlockSpec((_C, B, D), lambda i: (i, 0, 0)),
        out_shape=jax.ShapeDtypeStruct((L, B, D), jnp.float32),
        scratch_shapes=[
            pltpu.VMEM((L + 1, B, MSG), jnp.float32),
        ],
    )(plocal, hf, hb, wiou, biou, ucat, buf)


def kernel(embed_ids, sentence_len, parents, emb_matrix, W_ih_f, W_hh_f,
           b_ih_f, b_hh_f, W_ih_b, W_hh_b, b_ih_b, b_hh_b, W_iou, b_iou,
           U_iou, U_f, b_Uf, W_hid, b_hid):
    del sentence_len, W_hid, b_hid
    ids_tm = embed_ids.T.reshape(N).astype(jnp.int32)
    emb_pad = jnp.pad(emb_matrix, ((0, 0), (0, EMB_P - EMB)))
    embeds = _gather_embeddings(emb_pad, ids_tm).reshape(L, B, EMB_P)

    pad_w = ((0, EMB_P - EMB), (0, 0))
    wif = jnp.pad(W_ih_f.T, pad_w)
    whf = W_hh_f.T
    bf = (b_ih_f + b_hh_f).reshape(1, G4)
    wib = jnp.pad(W_ih_b.T, pad_w)
    whb = W_hh_b.T
    bb = (b_ih_b + b_hh_b).reshape(1, G4)
    hf, hb = _bilstm(embeds, wif, whf, bf, wib, whb, bb)

    base = (jnp.arange(N, dtype=jnp.int32) // L) * L
    plocal = jnp.minimum(parents.astype(jnp.int32) - base, L).reshape(B, L).T
    wiou = W_iou.T
    ucat = jnp.concatenate([U_iou.T, U_f.T], axis=1)
    h_tm = _tree_lstm(plocal, hf, hb, wiou, b_iou.reshape(1, IOU), ucat,
                      b_Uf.reshape(1, D))
    return h_tm.transpose(1, 0, 2).reshape(N, D)


# speedup vs baseline: 206.8056x; 1.7233x over previous
import functools

import jax
import jax.numpy as jnp
from jax import lax
from jax.experimental import pallas as pl
from jax.experimental.pallas import tpu as pltpu
from jax.experimental.pallas import tpu_sc as plsc

B, L, EMB, H, D = 16, 512, 300, 256, 256
EMB_P = 384
N = B * L
G4 = 4 * H
IOU = 3 * D
MSG = 4 * D
HIGHEST = lax.Precision.HIGHEST


_PR = 4000


def _pad_body(t_ref, o_ref):
    o_ref[...] = jnp.concatenate(
        [t_ref[...], jnp.zeros((_PR, EMB_P - EMB), jnp.float32)], axis=-1)


def _pad_table(table):
    v = table.shape[0]
    return pl.pallas_call(
        _pad_body,
        grid=(v // _PR,),
        in_specs=[pl.BlockSpec((_PR, EMB), lambda i: (i, 0))],
        out_specs=pl.BlockSpec((_PR, EMB_P), lambda i: (i, 0)),
        out_shape=jax.ShapeDtypeStruct((v, EMB_P), jnp.float32),
    )(table)


def _gather_embeddings(emb_matrix, ids_tm):
    info = plsc.get_sparse_core_info()
    nw = info.num_cores * info.num_subcores
    per_w = N // nw
    K = 128
    nk = per_w // K
    idx3 = ids_tm.reshape(nw, nk, K)
    mesh = plsc.VectorSubcoreMesh(core_axis_name="c", subcore_axis_name="s")

    @functools.partial(
        pl.kernel, mesh=mesh,
        out_type=jax.ShapeDtypeStruct((N, EMB_P), jnp.float32),
        scratch_types=[
            pltpu.VMEM((nk, K), jnp.int32),
            pltpu.VMEM((per_w, EMB_P), jnp.float32),
            pltpu.SemaphoreType.DMA,
        ],
    )
    def gather_k(table_hbm, idx_hbm, out_hbm, idx_v, rows_v, sem):
        wid = lax.axis_index("s") * info.num_cores + lax.axis_index("c")
        pltpu.sync_copy(idx_hbm.at[wid], idx_v)
        for j in range(nk):
            pltpu.async_copy(
                table_hbm.at[idx_v.at[j]], rows_v.at[pl.ds(j * K, K)], sem
            ).wait()
        pltpu.sync_copy(rows_v, out_hbm.at[pl.ds(wid * per_w, per_w)])

    return gather_k(emb_matrix, idx3)


_C = 64
_NT = L // _C


def _lstm_body(emb_f_ref, emb_b_ref, wif_ref, whf_ref, bf_ref,
               wib_ref, whb_ref, bb_ref, hf_out, hb_out,
               xf_ref, xb_ref, hf_c, cf_c, hb_c, cb_c):
    i = pl.program_id(0)

    @pl.when(i == 0)
    def _():
        hf_c[...] = jnp.zeros_like(hf_c)
        cf_c[...] = jnp.zeros_like(cf_c)
        hb_c[...] = jnp.zeros_like(hb_c)
        cb_c[...] = jnp.zeros_like(cb_c)

    ef = emb_f_ref[...].reshape(_C * B, EMB_P)
    xf_ref[...] = (jnp.dot(ef, wif_ref[...], preferred_element_type=jnp.float32)
                   + bf_ref[...]).reshape(_C, B, G4)
    eb = emb_b_ref[...].reshape(_C * B, EMB_P)
    xb_ref[...] = (jnp.dot(eb, wib_ref[...], preferred_element_type=jnp.float32)
                   + bb_ref[...]).reshape(_C, B, G4)

    def cell(g, c):
        ig = g[:, 0:H]
        fg = g[:, H:2 * H]
        gg = g[:, 2 * H:3 * H]
        og = g[:, 3 * H:4 * H]
        c_new = jax.nn.sigmoid(fg) * c + jax.nn.sigmoid(ig) * jnp.tanh(gg)
        h_new = jax.nn.sigmoid(og) * jnp.tanh(c_new)
        return h_new, c_new

    def step(j, carry):
        hf, cf, hb, cb = carry
        gf = xf_ref[j] + jnp.dot(hf, whf_ref[...], preferred_element_type=jnp.float32)
        hf, cf = cell(gf, cf)
        hf_out[j] = hf
        r = _C - 1 - j
        gb = xb_ref[r] + jnp.dot(hb, whb_ref[...], preferred_element_type=jnp.float32)
        hb, cb = cell(gb, cb)
        hb_out[r] = hb
        return (hf, cf, hb, cb)

    hf, cf, hb, cb = lax.fori_loop(
        0, _C, step, (hf_c[...], cf_c[...], hb_c[...], cb_c[...]))
    hf_c[...] = hf
    cf_c[...] = cf
    hb_c[...] = hb
    cb_c[...] = cb


def _bilstm(embeds, wif, whf, bf, wib, whb, bb):
    return pl.pallas_call(
        _lstm_body,
        grid=(_NT,),
        in_specs=[
            pl.BlockSpec((_C, B, EMB_P), lambda i: (i, 0, 0)),
            pl.BlockSpec((_C, B, EMB_P), lambda i: (_NT - 1 - i, 0, 0)),
            pl.BlockSpec((EMB_P, G4), lambda i: (0, 0)),
            pl.BlockSpec((H, G4), lambda i: (0, 0)),
            pl.BlockSpec((1, G4), lambda i: (0, 0)),
            pl.BlockSpec((EMB_P, G4), lambda i: (0, 0)),
            pl.BlockSpec((H, G4), lambda i: (0, 0)),
            pl.BlockSpec((1, G4), lambda i: (0, 0)),
        ],
        out_specs=[
            pl.BlockSpec((_C, B, H), lambda i: (i, 0, 0)),
            pl.BlockSpec((_C, B, H), lambda i: (_NT - 1 - i, 0, 0)),
        ],
        out_shape=[
            jax.ShapeDtypeStruct((L, B, H), jnp.float32),
            jax.ShapeDtypeStruct((L, B, H), jnp.float32),
        ],
        scratch_shapes=[
            pltpu.VMEM((_C, B, G4), jnp.float32),
            pltpu.VMEM((_C, B, G4), jnp.float32),
            pltpu.VMEM((B, H), jnp.float32),
            pltpu.VMEM((B, H), jnp.float32),
            pltpu.VMEM((B, H), jnp.float32),
            pltpu.VMEM((B, H), jnp.float32),
        ],
    )(embeds, embeds, wif, whf, bf, wib, whb, bb)


def _tree_body(plocal_ref, hf_ref, hb_ref, wiou_ref, biou_ref, ucat_ref,
               buf_ref, hout_ref, aci_ref, acc_ref):
    i = pl.program_id(0)

    @pl.when(i == 0)
    def _():
        aci_ref[...] = jnp.zeros_like(aci_ref)
        acc_ref[...] = jnp.zeros_like(acc_ref)

    x = jnp.concatenate([hf_ref[...], hb_ref[...]], axis=-1).reshape(_C * B, 2 * H)
    iou0 = (jnp.dot(x, wiou_ref[...], preferred_element_type=jnp.float32) + biou_ref[...]).reshape(_C, B, IOU)
    aci_ref[pl.ds(i * _C, _C), :, :] += iou0

    def step(j, _):
        v = i * _C + j
        iou = aci_ref[v]
        c_prev = acc_ref[v]
        ig = iou[:, 0:D]
        og = iou[:, D:2 * D]
        ug = iou[:, 2 * D:3 * D]
        c_v = jax.nn.sigmoid(ig) * jnp.tanh(ug) + c_prev
        h_v = jax.nn.sigmoid(og) * jnp.tanh(c_v)
        hout_ref[j] = h_v
        m = jnp.dot(h_v, ucat_ref[...], preferred_element_type=jnp.float32)
        f = jax.nn.sigmoid(m[:, IOU:MSG] + buf_ref[...])
        c_msg = f * c_v
        ps = [plocal_ref[v, b] for b in range(B)]
        cur_i = [aci_ref[ps[b], b:b + 1, :] for b in range(B)]
        cur_c = [acc_ref[ps[b], b:b + 1, :] for b in range(B)]
        for b in range(B):
            aci_ref[ps[b], b:b + 1, :] = cur_i[b] + m[b:b + 1, 0:IOU]
            acc_ref[ps[b], b:b + 1, :] = cur_c[b] + c_msg[b:b + 1, :]
        return 0

    lax.fori_loop(0, _C, step, 0)


def _tree_lstm(plocal, hf, hb, wiou, biou, ucat, buf):
    return pl.pallas_call(
        _tree_body,
        grid=(_NT,),
        in_specs=[
            pl.BlockSpec(memory_space=pltpu.SMEM),
            pl.BlockSpec((_C, B, H), lambda i: (i, 0, 0)),
            pl.BlockSpec((_C, B, H), lambda i: (i, 0, 0)),
            pl.BlockSpec((2 * H, IOU), lambda i: (0, 0)),
            pl.BlockSpec((1, IOU), lambda i: (0, 0)),
            pl.BlockSpec((D, MSG), lambda i: (0, 0)),
            pl.BlockSpec((1, D), lambda i: (0, 0)),
        ],
        out_specs=pl.BlockSpec((_C, B, D), lambda i: (i, 0, 0)),
        out_shape=jax.ShapeDtypeStruct((L, B, D), jnp.float32),
        scratch_shapes=[
            pltpu.VMEM((L + 1, B, IOU), jnp.float32),
            pltpu.VMEM((L + 1, B, D), jnp.float32),
        ],
    )(plocal, hf, hb, wiou, biou, ucat, buf)


def kernel(embed_ids, sentence_len, parents, emb_matrix, W_ih_f, W_hh_f,
           b_ih_f, b_hh_f, W_ih_b, W_hh_b, b_ih_b, b_hh_b, W_iou, b_iou,
           U_iou, U_f, b_Uf, W_hid, b_hid):
    del sentence_len, W_hid, b_hid
    ids_tm = embed_ids.T.reshape(N).astype(jnp.int32)
    emb_pad = _pad_table(emb_matrix)
    embeds = _gather_embeddings(emb_pad, ids_tm).reshape(L, B, EMB_P)

    pad_w = ((0, EMB_P - EMB), (0, 0))
    wif = jnp.pad(W_ih_f.T, pad_w)
    whf = W_hh_f.T
    bf = (b_ih_f + b_hh_f).reshape(1, G4)
    wib = jnp.pad(W_ih_b.T, pad_w)
    whb = W_hh_b.T
    bb = (b_ih_b + b_hh_b).reshape(1, G4)
    hf, hb = _bilstm(embeds, wif, whf, bf, wib, whb, bb)

    base = (jnp.arange(N, dtype=jnp.int32) // L) * L
    plocal = jnp.minimum(parents.astype(jnp.int32) - base, L).reshape(B, L).T
    wiou = W_iou.T
    ucat = jnp.concatenate([U_iou.T, U_f.T], axis=1)
    h_tm = _tree_lstm(plocal, hf, hb, wiou, b_iou.reshape(1, IOU), ucat,
                      b_Uf.reshape(1, D))
    return h_tm.transpose(1, 0, 2).reshape(N, D)
